# baseline (speedup 1.0000x reference)
import functools

import jax
import jax.numpy as jnp
from jax import lax
from jax.experimental import pallas as pl
from jax.experimental.pallas import tpu as pltpu
from jax.experimental.pallas import tpu_sc as plsc

N = 10000
E = 320000
D = 128
G = 128

NC = 2
NS = 16
NW = NC * NS
CHUNK = 128
CHUNKS_PER_TILE = 80
EDGES_PER_TILE = CHUNK * CHUNKS_PER_TILE
E_PAD = EDGES_PER_TILE * NW
GROUP = 16
C0_CHUNKS = 160
C1_CHUNKS = 0
CNT_CHUNKS = E_PAD // CHUNK // NW
ROWS_PER_TILE = 632
N_PAD = ROWS_PER_TILE * NS
CW = 16

R = 1000
NB = N // R


def _emit_gather_scatter(x_hbm, src_hbm, dst_hbm, acc,
                         src_v, dst_v, rows_a, rows_b, sem_a, sem_b,
                         tile_chunk0, n_chunks):
    for grp in range(n_chunks // GROUP):
        chunk0 = pl.multiple_of(tile_chunk0 + grp * GROUP, 8)
        pltpu.sync_copy(src_hbm.at[pl.ds(chunk0, GROUP)], src_v)
        pltpu.sync_copy(dst_hbm.at[pl.ds(chunk0, GROUP)], dst_v)
        pltpu.async_copy(x_hbm.at[src_v.at[0]], rows_a, sem_a)

        def pair_body(j, carry):
            i0 = j * 2
            pltpu.async_copy(x_hbm.at[src_v.at[i0 + 1]], rows_b, sem_b)
            pltpu.make_async_copy(x_hbm.at[src_v.at[i0]], rows_a,
                                  sem_a).wait()
            pltpu.sync_copy(rows_a, acc.at[dst_v.at[i0]], add=True)
            pltpu.async_copy(x_hbm.at[src_v.at[i0 + 2]], rows_a, sem_a)
            pltpu.make_async_copy(x_hbm.at[src_v.at[i0 + 1]], rows_b,
                                  sem_b).wait()
            pltpu.sync_copy(rows_b, acc.at[dst_v.at[i0 + 1]], add=True)
            return carry

        lax.fori_loop(0, GROUP // 2 - 1, pair_body, 0)
        last = GROUP - 2
        pltpu.async_copy(x_hbm.at[src_v.at[last + 1]], rows_b, sem_b)
        pltpu.make_async_copy(x_hbm.at[src_v.at[last]], rows_a,
                              sem_a).wait()
        pltpu.sync_copy(rows_a, acc.at[dst_v.at[last]], add=True)
        pltpu.make_async_copy(x_hbm.at[src_v.at[last + 1]], rows_b,
                              sem_b).wait()
        pltpu.sync_copy(rows_b, acc.at[dst_v.at[last + 1]], add=True)


@functools.lru_cache(maxsize=None)
def _make_seg_kernel():
    mesh = plsc.VectorSubcoreMesh(core_axis_name="c", subcore_axis_name="s",
                                  num_cores=NC, num_subcores=NS)
    out_type = jax.ShapeDtypeStruct((NC, N_PAD, D), jnp.float32)
    scratch = [
        pltpu.VMEM((GROUP, CHUNK), jnp.int32),
        pltpu.VMEM((GROUP, CHUNK), jnp.int32),
        pltpu.VMEM((CHUNK, D), jnp.float32),
        pltpu.VMEM((CHUNK, D), jnp.float32),
        pltpu.VMEM_SHARED((N_PAD, D), jnp.float32),
        pltpu.SemaphoreType.DMA,
        pltpu.SemaphoreType.DMA,
    ]

    @functools.partial(pl.kernel, out_type=out_type, mesh=mesh,
                       scratch_types=scratch)
    def seg(x_hbm, src_hbm, dst_hbm, zfeat_hbm, msg_hbm,
            src_v, dst_v, rows_a, rows_b, acc, sem_a, sem_b):
        c = lax.axis_index("c")
        s = lax.axis_index("s")
        row0 = pl.multiple_of(s * ROWS_PER_TILE, 8)
        pltpu.sync_copy(zfeat_hbm.at[pl.ds(row0, ROWS_PER_TILE)],
                        acc.at[pl.ds(row0, ROWS_PER_TILE)])
        plsc.subcore_barrier()

        @pl.when(c == 0)
        def _():
            _emit_gather_scatter(x_hbm, src_hbm, dst_hbm, acc,
                                 src_v, dst_v, rows_a, rows_b,
                                 sem_a, sem_b, s * C0_CHUNKS, C0_CHUNKS)

        if C1_CHUNKS:
            @pl.when(c == 1)
            def _():
                _emit_gather_scatter(x_hbm, src_hbm, dst_hbm, acc,
                                     src_v, dst_v, rows_a, rows_b,
                                     sem_a, sem_b,
                                     NS * C0_CHUNKS + s * C1_CHUNKS,
                                     C1_CHUNKS)

        plsc.subcore_barrier()
        pltpu.sync_copy(acc.at[pl.ds(row0, ROWS_PER_TILE)],
                        msg_hbm.at[c, pl.ds(row0, ROWS_PER_TILE)])

    return seg


@functools.lru_cache(maxsize=None)
def _make_cnt_kernel():
    mesh = plsc.VectorSubcoreMesh(core_axis_name="c", subcore_axis_name="s",
                                  num_cores=NC, num_subcores=NS)
    out_type = jax.ShapeDtypeStruct((NC, N_PAD, D), jnp.float32)
    scratch = [
        pltpu.VMEM((CNT_CHUNKS, CHUNK), jnp.int32),
        pltpu.VMEM((CHUNK, D), jnp.float32),
        pltpu.VMEM_SHARED((N_PAD, D), jnp.float32),
    ]

    @functools.partial(pl.kernel, out_type=out_type, mesh=mesh,
                       scratch_types=scratch)
    def cntk(dst_hbm, zfeat_hbm, ones_hbm, cnt_hbm, dst_v, ones_v, acc):
        c = lax.axis_index("c")
        s = lax.axis_index("s")
        w = c * NS + s
        row0 = pl.multiple_of(s * ROWS_PER_TILE, 8)
        pltpu.sync_copy(zfeat_hbm.at[pl.ds(row0, ROWS_PER_TILE)],
                        acc.at[pl.ds(row0, ROWS_PER_TILE)])
        pltpu.sync_copy(ones_hbm, ones_v)
        chunk0 = pl.multiple_of(w * CNT_CHUNKS, 8)
        pltpu.sync_copy(dst_hbm.at[pl.ds(chunk0, CNT_CHUNKS)], dst_v)
        plsc.subcore_barrier()

        def chunk_body(i, carry):
            pltpu.sync_copy(ones_v, acc.at[dst_v.at[i]], add=True)
            return carry

        lax.fori_loop(0, CNT_CHUNKS, chunk_body, 0)
        plsc.subcore_barrier()
        pltpu.sync_copy(acc.at[pl.ds(row0, ROWS_PER_TILE)],
                        cnt_hbm.at[c, pl.ds(row0, ROWS_PER_TILE)])

    return cntk


def _seg_plain(*args):
    return _make_seg_kernel()(*args)


def _cnt_call(*args):
    return _make_cnt_kernel()(*args)


def _sage_body(msg_ref, cnt_ref, x_ref, wl_ref, wr_ref, bl_ref, z_ref, st_ref):
    b = pl.program_id(0)
    msg = msg_ref[0] + msg_ref[1]
    cnt = cnt_ref[0, :, 0:1] + cnt_ref[1, :, 0:1]
    aggr = msg / jnp.maximum(cnt, 1.0)
    z = jnp.dot(aggr, wl_ref[...], preferred_element_type=jnp.float32)
    z = z + jnp.dot(x_ref[...], wr_ref[...], preferred_element_type=jnp.float32)
    z = jnp.maximum(z + bl_ref[...], 0.0)
    z_ref[...] = z

    @pl.when(b == 0)
    def _():
        st_ref[...] = jnp.zeros_like(st_ref)

    s1 = jnp.sum(z, axis=0, keepdims=True)
    s2 = jnp.sum(z * z, axis=0, keepdims=True)
    st_ref[...] += jnp.concatenate([s1, s2], axis=0)


def _sage_call(msg2, cnt2, x, wl, wr, bl):
    return pl.pallas_call(
        _sage_body,
        grid=(NB,),
        in_specs=[
            pl.BlockSpec((NC, R, D), lambda b: (0, b, 0)),
            pl.BlockSpec((NC, R, D), lambda b: (0, b, 0)),
            pl.BlockSpec((R, D), lambda b: (b, 0)),
            pl.BlockSpec((D, D), lambda b: (0, 0)),
            pl.BlockSpec((D, D), lambda b: (0, 0)),
            pl.BlockSpec((1, D), lambda b: (0, 0)),
        ],
        out_specs=[
            pl.BlockSpec((R, D), lambda b: (b, 0)),
            pl.BlockSpec((2, D), lambda b: (0, 0)),
        ],
        out_shape=[
            jax.ShapeDtypeStruct((N, D), jnp.float32),
            jax.ShapeDtypeStruct((2, D), jnp.float32),
        ],
    )(msg2, cnt2, x, wl, wr, bl)


def _gin_body(h_ref, sp_ref, w1_ref, b1_ref, w2_ref, b2_ref, z_ref, st_ref):
    b = pl.program_id(0)
    t = h_ref[...] + sp_ref[0] + sp_ref[1]
    t = jnp.dot(t, w1_ref[...], preferred_element_type=jnp.float32)
    t = jnp.maximum(t + b1_ref[...], 0.0)
    z = jnp.dot(t, w2_ref[...], preferred_element_type=jnp.float32)
    z = jnp.maximum(z + b2_ref[...], 0.0)
    z_ref[...] = z

    @pl.when(b == 0)
    def _():
        st_ref[...] = jnp.zeros_like(st_ref)

    s1 = jnp.sum(z, axis=0, keepdims=True)
    s2 = jnp.sum(z * z, axis=0, keepdims=True)
    st_ref[...] += jnp.concatenate([s1, s2], axis=0)


def _gin_call(h, sp, w1, b1, w2, b2):
    return pl.pallas_call(
        _gin_body,
        grid=(NB,),
        in_specs=[
            pl.BlockSpec((R, D), lambda b: (b, 0)),
            pl.BlockSpec((NC, R, D), lambda b: (0, b, 0)),
            pl.BlockSpec((D, D), lambda b: (0, 0)),
            pl.BlockSpec((1, D), lambda b: (0, 0)),
            pl.BlockSpec((D, D), lambda b: (0, 0)),
            pl.BlockSpec((1, D), lambda b: (0, 0)),
        ],
        out_specs=[
            pl.BlockSpec((R, D), lambda b: (b, 0)),
            pl.BlockSpec((2, D), lambda b: (0, 0)),
        ],
        out_shape=[
            jax.ShapeDtypeStruct((N, D), jnp.float32),
            jax.ShapeDtypeStruct((2, D), jnp.float32),
        ],
    )(h, sp, w1, b1, w2, b2)


def _bn_pool_body(z_ref, st_ref, g_ref, bb_ref, batch_ref, h_ref, p_ref):
    b = pl.program_id(0)
    st = st_ref[...]
    m = st[0:1, :] * (1.0 / N)
    var = st[1:2, :] * (1.0 / N) - m * m
    scale = g_ref[...] * lax.rsqrt(var + 1e-5)
    h = (z_ref[...] - m) * scale + bb_ref[...]
    h_ref[...] = h
    ids = batch_ref[0, 0, :]
    oh = (ids[:, None] == lax.broadcasted_iota(jnp.int32, (1, G), 1))
    oh = oh.astype(jnp.float32)
    p = lax.dot_general(oh, h, (((0,), (0,)), ((), ())),
                        preferred_element_type=jnp.float32)

    @pl.when(b == 0)
    def _():
        p_ref[...] = jnp.zeros_like(p_ref)

    p_ref[...] += p


def _bn_pool_call(z, st, g, bb, batch_r):
    return pl.pallas_call(
        _bn_pool_body,
        grid=(NB,),
        in_specs=[
            pl.BlockSpec((R, D), lambda b: (b, 0)),
            pl.BlockSpec((2, D), lambda b: (0, 0)),
            pl.BlockSpec((1, D), lambda b: (0, 0)),
            pl.BlockSpec((1, D), lambda b: (0, 0)),
            pl.BlockSpec((1, 1, R), lambda b: (b, 0, 0)),
        ],
        out_specs=[
            pl.BlockSpec((R, D), lambda b: (b, 0)),
            pl.BlockSpec((G, D), lambda b: (0, 0)),
        ],
        out_shape=[
            jax.ShapeDtypeStruct((N, D), jnp.float32),
            jax.ShapeDtypeStruct((G, D), jnp.float32),
        ],
    )(z, st, g, bb, batch_r)


def kernel(x, edge_index, batch, sage_Wl, sage_bl, sage_Wr,
           gin1_W1, gin1_b1, gin1_W2, gin1_b2,
           gin2_W1, gin2_b1, gin2_W2, gin2_b2,
           bn_g0, bn_b0, bn_g1, bn_b1, bn_g2, bn_b2):
    src = edge_index[0]
    dst = edge_index[1]
    npad = E_PAD - E
    src_p = jnp.concatenate([src, jnp.zeros((npad,), jnp.int32)])
    dst_p = jnp.concatenate([dst, jnp.full((npad,), N, jnp.int32)])
    src_p = src_p.reshape(E_PAD // CHUNK, CHUNK)
    dst_p = dst_p.reshape(E_PAD // CHUNK, CHUNK)
    zfeat = jnp.zeros((N_PAD, D), jnp.float32)
    ones128 = jnp.ones((CHUNK, D), jnp.float32)
    batch_r = batch.reshape(NB, 1, R)

    bl = sage_bl.reshape(1, D)
    b11 = gin1_b1.reshape(1, D)
    b12 = gin1_b2.reshape(1, D)
    b21 = gin2_b1.reshape(1, D)
    b22 = gin2_b2.reshape(1, D)
    g0, be0 = bn_g0.reshape(1, D), bn_b0.reshape(1, D)
    g1, be1 = bn_g1.reshape(1, D), bn_b1.reshape(1, D)
    g2, be2 = bn_g2.reshape(1, D), bn_b2.reshape(1, D)

    msg_p = _seg_plain(x, src_p, dst_p, zfeat)
    cnt_p = _cnt_call(dst_p, zfeat, ones128)
    z0, st0 = _sage_call(msg_p[:, :N], cnt_p[:, :N], x, sage_Wl, sage_Wr, bl)
    h0, p0 = _bn_pool_call(z0, st0, g0, be0, batch_r)

    s1_p = _seg_plain(h0, src_p, dst_p, zfeat)
    z1, st1 = _gin_call(h0, s1_p[:, :N], gin1_W1, b11, gin1_W2, b12)
    h1, p1 = _bn_pool_call(z1, st1, g1, be1, batch_r)

    s2_p = _seg_plain(h1, src_p, dst_p, zfeat)
    z2, st2 = _gin_call(h1, s2_p[:, :N], gin2_W1, b21, gin2_W2, b22)
    h2, p2 = _bn_pool_call(z2, st2, g2, be2, batch_r)

    pooled = jnp.concatenate([p0, p1, p2], axis=1)
    node_cat = jnp.concatenate([h0, h1, h2], axis=1)
    return (pooled, node_cat)


# speedup vs baseline: 1.4160x; 1.4160x over previous
import functools

import jax
import jax.numpy as jnp
from jax import lax
from jax.experimental import pallas as pl
from jax.experimental.pallas import tpu as pltpu
from jax.experimental.pallas import tpu_sc as plsc

N = 10000
E = 320000
D = 128
G = 128

NC = 2
NS = 16
NW = NC * NS
CHUNK = 128
CHUNKS_PER_TILE = 80
EDGES_PER_TILE = CHUNK * CHUNKS_PER_TILE
E_PAD = EDGES_PER_TILE * NW
GROUP = 8
C0_CHUNKS = 152
C1_CHUNKS = 8
CNT_CHUNKS = E_PAD // CHUNK // NW
ROWS_PER_TILE = 632
N_PAD = ROWS_PER_TILE * NS
CW = 16

R = 1000
NB = N // R


def _emit_gather_scatter(x_hbm, src_hbm, dst_hbm, acc,
                         src_v, dst_v, rows_a, rows_b, sem_a, sem_b,
                         tile_chunk0, n_chunks):
    for grp in range(n_chunks // GROUP):
        chunk0 = pl.multiple_of(tile_chunk0 + grp * GROUP, 8)
        pltpu.sync_copy(src_hbm.at[pl.ds(chunk0, GROUP)], src_v)
        pltpu.sync_copy(dst_hbm.at[pl.ds(chunk0, GROUP)], dst_v)
        pltpu.async_copy(x_hbm.at[src_v.at[0]], rows_a, sem_a)

        def pair_body(j, carry):
            i0 = j * 2
            pltpu.async_copy(x_hbm.at[src_v.at[i0 + 1]], rows_b, sem_b)
            pltpu.make_async_copy(x_hbm.at[src_v.at[i0]], rows_a,
                                  sem_a).wait()
            pltpu.sync_copy(rows_a, acc.at[dst_v.at[i0]], add=True)
            pltpu.async_copy(x_hbm.at[src_v.at[i0 + 2]], rows_a, sem_a)
            pltpu.make_async_copy(x_hbm.at[src_v.at[i0 + 1]], rows_b,
                                  sem_b).wait()
            pltpu.sync_copy(rows_b, acc.at[dst_v.at[i0 + 1]], add=True)
            return carry

        lax.fori_loop(0, GROUP // 2 - 1, pair_body, 0)
        last = GROUP - 2
        pltpu.async_copy(x_hbm.at[src_v.at[last + 1]], rows_b, sem_b)
        pltpu.make_async_copy(x_hbm.at[src_v.at[last]], rows_a,
                              sem_a).wait()
        pltpu.sync_copy(rows_a, acc.at[dst_v.at[last]], add=True)
        pltpu.make_async_copy(x_hbm.at[src_v.at[last + 1]], rows_b,
                              sem_b).wait()
        pltpu.sync_copy(rows_b, acc.at[dst_v.at[last + 1]], add=True)


@functools.lru_cache(maxsize=None)
def _make_seg_kernel():
    mesh = plsc.VectorSubcoreMesh(core_axis_name="c", subcore_axis_name="s",
                                  num_cores=NC, num_subcores=NS)
    out_type = jax.ShapeDtypeStruct((NC, N_PAD, D), jnp.float32)
    scratch = [
        pltpu.VMEM((GROUP, CHUNK), jnp.int32),
        pltpu.VMEM((GROUP, CHUNK), jnp.int32),
        pltpu.VMEM((CHUNK, D), jnp.float32),
        pltpu.VMEM((CHUNK, D), jnp.float32),
        pltpu.VMEM_SHARED((N_PAD, D), jnp.float32),
        pltpu.SemaphoreType.DMA,
        pltpu.SemaphoreType.DMA,
    ]

    @functools.partial(pl.kernel, out_type=out_type, mesh=mesh,
                       scratch_types=scratch)
    def seg(x_hbm, src_hbm, dst_hbm, zfeat_hbm, msg_hbm,
            src_v, dst_v, rows_a, rows_b, acc, sem_a, sem_b):
        c = lax.axis_index("c")
        s = lax.axis_index("s")
        row0 = pl.multiple_of(s * ROWS_PER_TILE, 8)
        pltpu.sync_copy(zfeat_hbm.at[pl.ds(row0, ROWS_PER_TILE)],
                        acc.at[pl.ds(row0, ROWS_PER_TILE)])
        plsc.subcore_barrier()

        @pl.when(c == 0)
        def _():
            _emit_gather_scatter(x_hbm, src_hbm, dst_hbm, acc,
                                 src_v, dst_v, rows_a, rows_b,
                                 sem_a, sem_b, s * C0_CHUNKS, C0_CHUNKS)

        if C1_CHUNKS:
            @pl.when(c == 1)
            def _():
                _emit_gather_scatter(x_hbm, src_hbm, dst_hbm, acc,
                                     src_v, dst_v, rows_a, rows_b,
                                     sem_a, sem_b,
                                     NS * C0_CHUNKS + s * C1_CHUNKS,
                                     C1_CHUNKS)

        plsc.subcore_barrier()
        pltpu.sync_copy(acc.at[pl.ds(row0, ROWS_PER_TILE)],
                        msg_hbm.at[c, pl.ds(row0, ROWS_PER_TILE)])

    return seg


@functools.lru_cache(maxsize=None)
def _make_cnt_kernel():
    mesh = plsc.VectorSubcoreMesh(core_axis_name="c", subcore_axis_name="s",
                                  num_cores=NC, num_subcores=NS)
    out_type = jax.ShapeDtypeStruct((NC, N_PAD, D), jnp.float32)
    scratch = [
        pltpu.VMEM((CNT_CHUNKS, CHUNK), jnp.int32),
        pltpu.VMEM((CHUNK, D), jnp.float32),
        pltpu.VMEM_SHARED((N_PAD, D), jnp.float32),
    ]

    @functools.partial(pl.kernel, out_type=out_type, mesh=mesh,
                       scratch_types=scratch)
    def cntk(dst_hbm, zfeat_hbm, ones_hbm, cnt_hbm, dst_v, ones_v, acc):
        c = lax.axis_index("c")
        s = lax.axis_index("s")
        w = c * NS + s
        row0 = pl.multiple_of(s * ROWS_PER_TILE, 8)
        pltpu.sync_copy(zfeat_hbm.at[pl.ds(row0, ROWS_PER_TILE)],
                        acc.at[pl.ds(row0, ROWS_PER_TILE)])
        pltpu.sync_copy(ones_hbm, ones_v)
        chunk0 = pl.multiple_of(w * CNT_CHUNKS, 8)
        pltpu.sync_copy(dst_hbm.at[pl.ds(chunk0, CNT_CHUNKS)], dst_v)
        plsc.subcore_barrier()

        def chunk_body(i, carry):
            pltpu.sync_copy(ones_v, acc.at[dst_v.at[i]], add=True)
            return carry

        lax.fori_loop(0, CNT_CHUNKS, chunk_body, 0)
        plsc.subcore_barrier()
        pltpu.sync_copy(acc.at[pl.ds(row0, ROWS_PER_TILE)],
                        cnt_hbm.at[c, pl.ds(row0, ROWS_PER_TILE)])

    return cntk


def _seg_plain(*args):
    return _make_seg_kernel()(*args)


def _cnt_call(*args):
    return _make_cnt_kernel()(*args)


def _sage_body(msg_ref, cnt_ref, x_ref, wl_ref, wr_ref, bl_ref, z_ref, st_ref):
    b = pl.program_id(0)
    msg = msg_ref[0] + msg_ref[1]
    cnt = cnt_ref[0, :, 0:1] + cnt_ref[1, :, 0:1]
    aggr = msg / jnp.maximum(cnt, 1.0)
    z = jnp.dot(aggr, wl_ref[...], preferred_element_type=jnp.float32)
    z = z + jnp.dot(x_ref[...], wr_ref[...], preferred_element_type=jnp.float32)
    z = jnp.maximum(z + bl_ref[...], 0.0)
    z_ref[...] = z

    @pl.when(b == 0)
    def _():
        st_ref[...] = jnp.zeros_like(st_ref)

    s1 = jnp.sum(z, axis=0, keepdims=True)
    s2 = jnp.sum(z * z, axis=0, keepdims=True)
    st_ref[...] += jnp.concatenate([s1, s2], axis=0)


def _sage_call(msg2, cnt2, x, wl, wr, bl):
    return pl.pallas_call(
        _sage_body,
        grid=(NB,),
        in_specs=[
            pl.BlockSpec((NC, R, D), lambda b: (0, b, 0)),
            pl.BlockSpec((NC, R, D), lambda b: (0, b, 0)),
            pl.BlockSpec((R, D), lambda b: (b, 0)),
            pl.BlockSpec((D, D), lambda b: (0, 0)),
            pl.BlockSpec((D, D), lambda b: (0, 0)),
            pl.BlockSpec((1, D), lambda b: (0, 0)),
        ],
        out_specs=[
            pl.BlockSpec((R, D), lambda b: (b, 0)),
            pl.BlockSpec((2, D), lambda b: (0, 0)),
        ],
        out_shape=[
            jax.ShapeDtypeStruct((N, D), jnp.float32),
            jax.ShapeDtypeStruct((2, D), jnp.float32),
        ],
    )(msg2, cnt2, x, wl, wr, bl)


def _gin_body(h_ref, sp_ref, w1_ref, b1_ref, w2_ref, b2_ref, z_ref, st_ref):
    b = pl.program_id(0)
    t = h_ref[...] + sp_ref[0] + sp_ref[1]
    t = jnp.dot(t, w1_ref[...], preferred_element_type=jnp.float32)
    t = jnp.maximum(t + b1_ref[...], 0.0)
    z = jnp.dot(t, w2_ref[...], preferred_element_type=jnp.float32)
    z = jnp.maximum(z + b2_ref[...], 0.0)
    z_ref[...] = z

    @pl.when(b == 0)
    def _():
        st_ref[...] = jnp.zeros_like(st_ref)

    s1 = jnp.sum(z, axis=0, keepdims=True)
    s2 = jnp.sum(z * z, axis=0, keepdims=True)
    st_ref[...] += jnp.concatenate([s1, s2], axis=0)


def _gin_call(h, sp, w1, b1, w2, b2):
    return pl.pallas_call(
        _gin_body,
        grid=(NB,),
        in_specs=[
            pl.BlockSpec((R, D), lambda b: (b, 0)),
            pl.BlockSpec((NC, R, D), lambda b: (0, b, 0)),
            pl.BlockSpec((D, D), lambda b: (0, 0)),
            pl.BlockSpec((1, D), lambda b: (0, 0)),
            pl.BlockSpec((D, D), lambda b: (0, 0)),
            pl.BlockSpec((1, D), lambda b: (0, 0)),
        ],
        out_specs=[
            pl.BlockSpec((R, D), lambda b: (b, 0)),
            pl.BlockSpec((2, D), lambda b: (0, 0)),
        ],
        out_shape=[
            jax.ShapeDtypeStruct((N, D), jnp.float32),
            jax.ShapeDtypeStruct((2, D), jnp.float32),
        ],
    )(h, sp, w1, b1, w2, b2)


def _bn_pool_body(z_ref, st_ref, g_ref, bb_ref, batch_ref, h_ref, p_ref):
    b = pl.program_id(0)
    st = st_ref[...]
    m = st[0:1, :] * (1.0 / N)
    var = st[1:2, :] * (1.0 / N) - m * m
    scale = g_ref[...] * lax.rsqrt(var + 1e-5)
    h = (z_ref[...] - m) * scale + bb_ref[...]
    h_ref[...] = h
    ids = batch_ref[0, 0, :]
    oh = (ids[:, None] == lax.broadcasted_iota(jnp.int32, (1, G), 1))
    oh = oh.astype(jnp.float32)
    p = lax.dot_general(oh, h, (((0,), (0,)), ((), ())),
                        preferred_element_type=jnp.float32)

    @pl.when(b == 0)
    def _():
        p_ref[...] = jnp.zeros_like(p_ref)

    p_ref[...] += p


def _bn_pool_call(z, st, g, bb, batch_r):
    return pl.pallas_call(
        _bn_pool_body,
        grid=(NB,),
        in_specs=[
            pl.BlockSpec((R, D), lambda b: (b, 0)),
            pl.BlockSpec((2, D), lambda b: (0, 0)),
            pl.BlockSpec((1, D), lambda b: (0, 0)),
            pl.BlockSpec((1, D), lambda b: (0, 0)),
            pl.BlockSpec((1, 1, R), lambda b: (b, 0, 0)),
        ],
        out_specs=[
            pl.BlockSpec((R, D), lambda b: (b, 0)),
            pl.BlockSpec((G, D), lambda b: (0, 0)),
        ],
        out_shape=[
            jax.ShapeDtypeStruct((N, D), jnp.float32),
            jax.ShapeDtypeStruct((G, D), jnp.float32),
        ],
    )(z, st, g, bb, batch_r)


def kernel(x, edge_index, batch, sage_Wl, sage_bl, sage_Wr,
           gin1_W1, gin1_b1, gin1_W2, gin1_b2,
           gin2_W1, gin2_b1, gin2_W2, gin2_b2,
           bn_g0, bn_b0, bn_g1, bn_b1, bn_g2, bn_b2):
    src = edge_index[0]
    dst = edge_index[1]
    npad = E_PAD - E
    src_p = jnp.concatenate([src, jnp.zeros((npad,), jnp.int32)])
    dst_p = jnp.concatenate([dst, jnp.full((npad,), N, jnp.int32)])
    src_p = src_p.reshape(E_PAD // CHUNK, CHUNK)
    dst_p = dst_p.reshape(E_PAD // CHUNK, CHUNK)
    zfeat = jnp.zeros((N_PAD, D), jnp.float32)
    ones128 = jnp.ones((CHUNK, D), jnp.float32)
    batch_r = batch.reshape(NB, 1, R)

    bl = sage_bl.reshape(1, D)
    b11 = gin1_b1.reshape(1, D)
    b12 = gin1_b2.reshape(1, D)
    b21 = gin2_b1.reshape(1, D)
    b22 = gin2_b2.reshape(1, D)
    g0, be0 = bn_g0.reshape(1, D), bn_b0.reshape(1, D)
    g1, be1 = bn_g1.reshape(1, D), bn_b1.reshape(1, D)
    g2, be2 = bn_g2.reshape(1, D), bn_b2.reshape(1, D)

    msg_p = _seg_plain(x, src_p, dst_p, zfeat)
    cnt_p = _cnt_call(dst_p, zfeat, ones128)
    z0, st0 = _sage_call(msg_p[:, :N], cnt_p[:, :N], x, sage_Wl, sage_Wr, bl)
    h0, p0 = _bn_pool_call(z0, st0, g0, be0, batch_r)

    s1_p = _seg_plain(h0, src_p, dst_p, zfeat)
    z1, st1 = _gin_call(h0, s1_p[:, :N], gin1_W1, b11, gin1_W2, b12)
    h1, p1 = _bn_pool_call(z1, st1, g1, be1, batch_r)

    s2_p = _seg_plain(h1, src_p, dst_p, zfeat)
    z2, st2 = _gin_call(h1, s2_p[:, :N], gin2_W1, b21, gin2_W2, b22)
    h2, p2 = _bn_pool_call(z2, st2, g2, be2, batch_r)

    pooled = jnp.concatenate([p0, p1, p2], axis=1)
    node_cat = jnp.concatenate([h0, h1, h2], axis=1)
    return (pooled, node_cat)
